# baseline (speedup 1.0000x reference)
import functools

import jax
import jax.numpy as jnp
from jax import lax
from jax.experimental import pallas as pl
from jax.experimental.pallas import tpu as pltpu
from jax.experimental.pallas import tpu_sc as plsc

DIM = 64
T = 50
B = 4096
CHUNKS = (256, 768, 1024, 1024, 768, 256)
BB = 32
RP = BB * T // 2
NW = 32


def _sc_gather(table, idx2d, chunk, kg, supers_per_worker):
  n = idx2d.shape[0] * 128
  super_ = 2 * chunk
  per_w = supers_per_worker * super_
  assert per_w * NW == n
  mesh = plsc.VectorSubcoreMesh(core_axis_name="c", subcore_axis_name="s")

  @functools.partial(
      pl.kernel,
      mesh=mesh,
      compiler_params=pltpu.CompilerParams(use_tc_tiling_on_sc=False),
      out_type=jax.ShapeDtypeStruct((n, DIM), jnp.float32),
      scratch_types=[
          pltpu.VMEM((2 * kg, 128), jnp.int32),
          pltpu.VMEM((chunk, DIM), jnp.float32),
          pltpu.VMEM((chunk, DIM), jnp.float32),
          pltpu.SemaphoreType.DMA,
          pltpu.SemaphoreType.DMA,
          pltpu.SemaphoreType.DMA,
          pltpu.SemaphoreType.DMA,
          pltpu.SemaphoreType.DMA,
      ],
  )
  def gk(table_hbm, idx_hbm, out_hbm, idx_v, rb0, rb1, g0, g1, w0, w1, isem):
    wid = lax.axis_index("s") * 2 + lax.axis_index("c")
    base = wid * per_w
    base128 = wid * (per_w // 128)
    bufs = ((rb0, g0, w0), (rb1, g1, w1))

    pltpu.sync_copy(idx_hbm.at[pl.ds(base128, 2 * kg)], idx_v)

    def super_body(s, carry):
      off = base + s * super_

      @pl.when(s > 0)
      def _wait_idx():
        pltpu.make_async_copy(
            idx_hbm.at[pl.ds(base128 + s * 2 * kg, 2 * kg)], idx_v,
            isem).wait()

      for b, (rb, gs, ws) in enumerate(bufs):
        coff = off + b * chunk

        @pl.when(s > 0)
        def _wait_wo(rb=rb, ws=ws, coff=coff):
          pltpu.make_async_copy(
              rb, out_hbm.at[pl.ds(coff - super_, chunk)], ws).wait()

        for j in range(kg):
          pltpu.async_copy(table_hbm.at[idx_v.at[b * kg + j]],
                           rb.at[pl.ds(j * 128, 128)], gs)

      for b, (rb, gs, ws) in enumerate(bufs):
        coff = off + b * chunk
        for j in range(kg):
          pltpu.make_async_copy(table_hbm.at[idx_v.at[b * kg + j]],
                                rb.at[pl.ds(j * 128, 128)], gs).wait()
        pltpu.async_copy(rb, out_hbm.at[pl.ds(coff, chunk)], ws)

      @pl.when(s + 1 < supers_per_worker)
      def _prefetch_idx():
        pltpu.async_copy(
            idx_hbm.at[pl.ds(base128 + (s + 1) * 2 * kg, 2 * kg)], idx_v,
            isem)

      return carry

    lax.fori_loop(0, supers_per_worker, super_body, 0)

    last = base + (supers_per_worker - 1) * super_
    pltpu.make_async_copy(rb0, out_hbm.at[pl.ds(last, chunk)], w0).wait()
    pltpu.make_async_copy(rb1, out_hbm.at[pl.ds(last + chunk, chunk)],
                          w1).wait()

  return gk(table, idx2d)


def _sc_gather_items(table, idx2d):
  mesh = plsc.VectorSubcoreMesh(core_axis_name="c", subcore_axis_name="s")

  @functools.partial(
      pl.kernel,
      mesh=mesh,
      compiler_params=pltpu.CompilerParams(use_tc_tiling_on_sc=False),
      out_type=jax.ShapeDtypeStruct((B, DIM), jnp.float32),
      scratch_types=[
          pltpu.VMEM((1, 128), jnp.int32),
          pltpu.VMEM((128, DIM), jnp.float32),
          pltpu.SemaphoreType.DMA,
      ],
  )
  def gk(table_hbm, idx_hbm, out_hbm, idx_v, rows_v, sem):
    wid = lax.axis_index("s") * 2 + lax.axis_index("c")
    pltpu.sync_copy(idx_hbm.at[pl.ds(wid, 1)], idx_v)
    pltpu.async_copy(table_hbm.at[idx_v.at[0]], rows_v, sem).wait()
    pltpu.sync_copy(rows_v, out_hbm.at[pl.ds(wid * 128, 128)])

  return gk(table, idx2d)


def _tc_body(*refs):
  e = refs[0:16]
  r = refs[16:24]
  items_ref, w1a2_ref, w1b2_ref, w2b_ref, out_ref = refs[24:29]

  w1a2 = w1a2_ref[...]
  w1b2 = w1b2_ref[...]
  w2b = w2b_ref[...]

  hp = T // 2
  rows = lax.broadcasted_iota(jnp.int32, (RP, BB), 0)
  cols = lax.broadcasted_iota(jnp.int32, (RP, BB), 1)
  m2 = jnp.where((rows // hp) == cols, 1.0, 0.0).astype(jnp.float32)

  def segsum(x):
    return lax.dot_general(m2, x, (((0,), (0,)), ((), ())),
                           preferred_element_type=jnp.float32)

  def fold(x):
    return x[:, :DIM] + x[:, DIM:]

  def attention(h2, p2, t2):
    s1 = jax.nn.sigmoid(
        jnp.dot(h2, w1a2, preferred_element_type=jnp.float32)
        + jnp.dot(p2, w1b2, preferred_element_type=jnp.float32))
    att2 = jax.nn.sigmoid(jnp.dot(s1, w2b,
                                  preferred_element_type=jnp.float32))
    e2 = jnp.exp(att2)
    eb = jnp.concatenate(
        [jnp.broadcast_to(e2[:, 0:1], (RP, DIM)),
         jnp.broadcast_to(e2[:, 1:2], (RP, DIM))], axis=1)
    num = fold(segsum(eb * t2))
    den2 = segsum(e2)
    den = den2[:, 0:1] + den2[:, 1:2]
    return num / den

  per_set = []
  for s in range(4):
    g00, g01, g20, g21 = (x[...] for x in e[4 * s:4 * s + 4])
    g10, g11 = (x[...] for x in r[2 * s:2 * s + 2])
    o0 = attention(g00, g10, g20)
    o1 = attention(g00 + g01, g10 * g11, g21)
    mean0 = fold(segsum(g00)) * (1.0 / T)
    per_set.append((mean0, o0, o1))

  u = per_set[0][0] + per_set[0][1] + per_set[0][2]
  ipx = per_set[1][0] + per_set[1][1] + per_set[1][2]
  up = per_set[2][0] + per_set[2][1] + per_set[2][2]
  io = per_set[3][0] + per_set[3][1] + per_set[3][2]

  base = jnp.sum(u * io + up * ipx, axis=1, keepdims=True)

  jrows = lax.broadcasted_iota(jnp.int32, (BB // 2, BB), 0)
  jcols = lax.broadcasted_iota(jnp.int32, (BB // 2, BB), 1)
  se = jnp.where(jcols == 2 * jrows, 1.0, 0.0).astype(jnp.float32)
  so = jnp.where(jcols == 2 * jrows + 1, 1.0, 0.0).astype(jnp.float32)

  def sel(mat, x):
    return lax.dot_general(mat, x, (((1,), (0,)), ((), ())),
                           preferred_element_type=jnp.float32)

  up_p = jnp.concatenate([sel(se, up), sel(so, up)], axis=1)
  prod = items_ref[...] * up_p
  extra_e = jnp.sum(prod[:, :DIM], axis=1, keepdims=True)
  extra_o = jnp.sum(prod[:, DIM:], axis=1, keepdims=True)
  score = jax.nn.sigmoid(jnp.concatenate(
      [sel(se, base) + extra_e, sel(so, base) + extra_o], axis=1))
  out_ref[...] = score


def _tc_attention(e_rows, r_rows, items_p, w1a2, w1b2, w2b, bc):
  pblocks = bc * T // 2 // RP
  in_specs = []
  for p in range(16):
    in_specs.append(pl.BlockSpec(
        (RP, 2 * DIM), lambda i, b=p * pblocks: (b + i, 0)))
  for p in range(8):
    in_specs.append(pl.BlockSpec(
        (RP, 2 * DIM), lambda i, b=p * pblocks: (b + i, 0)))
  in_specs.append(pl.BlockSpec((BB // 2, 2 * DIM), lambda i: (i, 0)))
  in_specs.append(pl.BlockSpec((2 * DIM, 2 * DIM), lambda i: (0, 0)))
  in_specs.append(pl.BlockSpec((2 * DIM, 2 * DIM), lambda i: (0, 0)))
  in_specs.append(pl.BlockSpec((2 * DIM, 2), lambda i: (0, 0)))

  return pl.pallas_call(
      _tc_body,
      grid=(bc // BB,),
      in_specs=in_specs,
      out_specs=pl.BlockSpec((BB // 2, 2), lambda i: (i, 0)),
      out_shape=jax.ShapeDtypeStruct((bc // 2, 2), jnp.float32),
  )(*([e_rows] * 16), *([r_rows] * 8), items_p, w1a2, w1b2, w2b)


def kernel(items, user_init_triple_set, item_potential_triple_set,
           user_potential_triple_set, item_origin_triple_set,
           entity_emb, relation_emb, W1, W2):
  sets = (user_init_triple_set, item_potential_triple_set,
          user_potential_triple_set, item_origin_triple_set)

  z = jnp.zeros((DIM, DIM), jnp.float32)
  w1a, w1b = W1[:DIM], W1[DIM:]
  w1a2 = jnp.concatenate(
      [jnp.concatenate([w1a, z], 1), jnp.concatenate([z, w1a], 1)], 0)
  w1b2 = jnp.concatenate(
      [jnp.concatenate([w1b, z], 1), jnp.concatenate([z, w1b], 1)], 0)
  zc = jnp.zeros((DIM, 1), jnp.float32)
  w2b = jnp.concatenate([jnp.concatenate([W2, zc], 0),
                         jnp.concatenate([zc, W2], 0)], 1)

  idt = sets[0].dtype
  items_rows = _sc_gather_items(entity_emb,
                                items.astype(idt).reshape(-1, 128))
  items_paired = items_rows.reshape(-1, 2 * DIM)

  offs = [0]
  for bc in CHUNKS:
    offs.append(offs[-1] + bc)

  e_chunk_idx = []
  r_chunk_idx = []
  for c, bc in enumerate(CHUNKS):
    sl = slice(offs[c], offs[c + 1])
    e_parts = []
    r_parts = []
    for ts in sets:
      e_parts += [ts[0, 0, sl].reshape(-1), ts[0, 1, sl].reshape(-1),
                  ts[2, 0, sl].reshape(-1), ts[2, 1, sl].reshape(-1)]
      r_parts += [ts[1, 0, sl].reshape(-1), ts[1, 1, sl].reshape(-1)]
    e_chunk_idx.append(jnp.concatenate(e_parts).reshape(-1, 128))
    r_chunk_idx.append(jnp.concatenate(r_parts).reshape(-1, 128))

  def gather_padded(table, idx2d, chunk):
    n = idx2d.shape[0] * 128
    gran = NW * 2 * chunk
    sup = -(-n // gran)
    npad = sup * gran - n
    if npad:
      idx2d = jnp.concatenate(
          [idx2d, jnp.zeros((npad // 128, 128), idx2d.dtype)])
    return _sc_gather(table, idx2d, chunk, chunk // 128, sup)

  e_rows = []
  r_rows = []
  for c, bc in enumerate(CHUNKS):
    ech = 512 if bc >= 1024 else (256 if bc >= 512 else 128)
    rch = 256 if bc >= 1024 else 128
    e_rows.append(gather_padded(entity_emb, e_chunk_idx[c], ech))
    r_rows.append(gather_padded(relation_emb, r_chunk_idx[c], rch))

  outs = []
  for c, bc in enumerate(CHUNKS):
    items_c = lax.dynamic_slice_in_dim(items_paired, offs[c] // 2,
                                       bc // 2, 0)
    outs.append(_tc_attention(e_rows[c].reshape(-1, 2 * DIM),
                              r_rows[c].reshape(-1, 2 * DIM),
                              items_c, w1a2, w1b2, w2b, bc))
  return jnp.concatenate(outs).reshape(B)


# speedup vs baseline: 1.4980x; 1.4980x over previous
import functools

import jax
import jax.numpy as jnp
from jax import lax
from jax.experimental import pallas as pl
from jax.experimental.pallas import tpu as pltpu
from jax.experimental.pallas import tpu_sc as plsc

DIM = 64
T = 50
B = 4096
CHUNKS = (512, 1024, 1024, 1024, 512)
BB = 32
RP = BB * T // 2
NW = 32


def _sc_gather(table, idx2d, chunk, kg, supers_per_worker):
  n = idx2d.shape[0] * 128
  super_ = 2 * chunk
  per_w = supers_per_worker * super_
  assert per_w * NW == n
  mesh = plsc.VectorSubcoreMesh(core_axis_name="c", subcore_axis_name="s")

  @functools.partial(
      pl.kernel,
      mesh=mesh,
      compiler_params=pltpu.CompilerParams(use_tc_tiling_on_sc=False),
      out_type=jax.ShapeDtypeStruct((n, DIM), jnp.float32),
      scratch_types=[
          pltpu.VMEM((2 * kg, 128), jnp.int32),
          pltpu.VMEM((chunk, DIM), jnp.float32),
          pltpu.VMEM((chunk, DIM), jnp.float32),
          pltpu.SemaphoreType.DMA,
          pltpu.SemaphoreType.DMA,
          pltpu.SemaphoreType.DMA,
          pltpu.SemaphoreType.DMA,
          pltpu.SemaphoreType.DMA,
      ],
  )
  def gk(table_hbm, idx_hbm, out_hbm, idx_v, rb0, rb1, g0, g1, w0, w1, isem):
    wid = lax.axis_index("s") * 2 + lax.axis_index("c")
    base = wid * per_w
    base128 = wid * (per_w // 128)
    bufs = ((rb0, g0, w0), (rb1, g1, w1))

    pltpu.sync_copy(idx_hbm.at[pl.ds(base128, 2 * kg)], idx_v)

    def super_body(s, carry):
      off = base + s * super_

      @pl.when(s > 0)
      def _wait_idx():
        pltpu.make_async_copy(
            idx_hbm.at[pl.ds(base128 + s * 2 * kg, 2 * kg)], idx_v,
            isem).wait()

      for b, (rb, gs, ws) in enumerate(bufs):
        coff = off + b * chunk

        @pl.when(s > 0)
        def _wait_wo(rb=rb, ws=ws, coff=coff):
          pltpu.make_async_copy(
              rb, out_hbm.at[pl.ds(coff - super_, chunk)], ws).wait()

        for j in range(kg):
          pltpu.async_copy(table_hbm.at[idx_v.at[b * kg + j]],
                           rb.at[pl.ds(j * 128, 128)], gs)

      for b, (rb, gs, ws) in enumerate(bufs):
        coff = off + b * chunk
        for j in range(kg):
          pltpu.make_async_copy(table_hbm.at[idx_v.at[b * kg + j]],
                                rb.at[pl.ds(j * 128, 128)], gs).wait()
        pltpu.async_copy(rb, out_hbm.at[pl.ds(coff, chunk)], ws)

      @pl.when(s + 1 < supers_per_worker)
      def _prefetch_idx():
        pltpu.async_copy(
            idx_hbm.at[pl.ds(base128 + (s + 1) * 2 * kg, 2 * kg)], idx_v,
            isem)

      return carry

    lax.fori_loop(0, supers_per_worker, super_body, 0)

    last = base + (supers_per_worker - 1) * super_
    pltpu.make_async_copy(rb0, out_hbm.at[pl.ds(last, chunk)], w0).wait()
    pltpu.make_async_copy(rb1, out_hbm.at[pl.ds(last + chunk, chunk)],
                          w1).wait()

  return gk(table, idx2d)


def _sc_gather_items(table, idx2d):
  mesh = plsc.VectorSubcoreMesh(core_axis_name="c", subcore_axis_name="s")

  @functools.partial(
      pl.kernel,
      mesh=mesh,
      compiler_params=pltpu.CompilerParams(use_tc_tiling_on_sc=False),
      out_type=jax.ShapeDtypeStruct((B, DIM), jnp.float32),
      scratch_types=[
          pltpu.VMEM((1, 128), jnp.int32),
          pltpu.VMEM((128, DIM), jnp.float32),
          pltpu.SemaphoreType.DMA,
      ],
  )
  def gk(table_hbm, idx_hbm, out_hbm, idx_v, rows_v, sem):
    wid = lax.axis_index("s") * 2 + lax.axis_index("c")
    pltpu.sync_copy(idx_hbm.at[pl.ds(wid, 1)], idx_v)
    pltpu.async_copy(table_hbm.at[idx_v.at[0]], rows_v, sem).wait()
    pltpu.sync_copy(rows_v, out_hbm.at[pl.ds(wid * 128, 128)])

  return gk(table, idx2d)


def _tc_body(*refs):
  e = refs[0:16]
  r = refs[16:24]
  items_ref, w1a2_ref, w1b2_ref, w2b_ref, out_ref = refs[24:29]

  w1a2 = w1a2_ref[...]
  w1b2 = w1b2_ref[...]
  w2b = w2b_ref[...]

  hp = T // 2
  rows = lax.broadcasted_iota(jnp.int32, (RP, BB), 0)
  cols = lax.broadcasted_iota(jnp.int32, (RP, BB), 1)
  m2 = jnp.where((rows // hp) == cols, 1.0, 0.0).astype(jnp.float32)

  def segsum(x):
    return lax.dot_general(m2, x, (((0,), (0,)), ((), ())),
                           preferred_element_type=jnp.float32)

  def fold(x):
    return x[:, :DIM] + x[:, DIM:]

  def attention(h2, p2, t2):
    s1 = jax.nn.sigmoid(
        jnp.dot(h2, w1a2, preferred_element_type=jnp.float32)
        + jnp.dot(p2, w1b2, preferred_element_type=jnp.float32))
    att2 = jax.nn.sigmoid(jnp.dot(s1, w2b,
                                  preferred_element_type=jnp.float32))
    e2 = jnp.exp(att2)
    eb = jnp.concatenate(
        [jnp.broadcast_to(e2[:, 0:1], (RP, DIM)),
         jnp.broadcast_to(e2[:, 1:2], (RP, DIM))], axis=1)
    num = fold(segsum(eb * t2))
    den2 = segsum(e2)
    den = den2[:, 0:1] + den2[:, 1:2]
    return num / den

  per_set = []
  for s in range(4):
    g00, g01, g20, g21 = (x[...] for x in e[4 * s:4 * s + 4])
    g10, g11 = (x[...] for x in r[2 * s:2 * s + 2])
    o0 = attention(g00, g10, g20)
    o1 = attention(g00 + g01, g10 * g11, g21)
    mean0 = fold(segsum(g00)) * (1.0 / T)
    per_set.append((mean0, o0, o1))

  u = per_set[0][0] + per_set[0][1] + per_set[0][2]
  ipx = per_set[1][0] + per_set[1][1] + per_set[1][2]
  up = per_set[2][0] + per_set[2][1] + per_set[2][2]
  io = per_set[3][0] + per_set[3][1] + per_set[3][2]

  base = jnp.sum(u * io + up * ipx, axis=1, keepdims=True)

  jrows = lax.broadcasted_iota(jnp.int32, (BB // 2, BB), 0)
  jcols = lax.broadcasted_iota(jnp.int32, (BB // 2, BB), 1)
  se = jnp.where(jcols == 2 * jrows, 1.0, 0.0).astype(jnp.float32)
  so = jnp.where(jcols == 2 * jrows + 1, 1.0, 0.0).astype(jnp.float32)

  def sel(mat, x):
    return lax.dot_general(mat, x, (((1,), (0,)), ((), ())),
                           preferred_element_type=jnp.float32)

  up_p = jnp.concatenate([sel(se, up), sel(so, up)], axis=1)
  prod = items_ref[...] * up_p
  extra_e = jnp.sum(prod[:, :DIM], axis=1, keepdims=True)
  extra_o = jnp.sum(prod[:, DIM:], axis=1, keepdims=True)
  score = jax.nn.sigmoid(jnp.concatenate(
      [sel(se, base) + extra_e, sel(so, base) + extra_o], axis=1))
  out_ref[...] = score


def _tc_attention(e_rows, r_rows, items_p, w1a2, w1b2, w2b, bc):
  pblocks = bc * T // 2 // RP
  in_specs = []
  for p in range(16):
    in_specs.append(pl.BlockSpec(
        (RP, 2 * DIM), lambda i, b=p * pblocks: (b + i, 0)))
  for p in range(8):
    in_specs.append(pl.BlockSpec(
        (RP, 2 * DIM), lambda i, b=p * pblocks: (b + i, 0)))
  in_specs.append(pl.BlockSpec((BB // 2, 2 * DIM), lambda i: (i, 0)))
  in_specs.append(pl.BlockSpec((2 * DIM, 2 * DIM), lambda i: (0, 0)))
  in_specs.append(pl.BlockSpec((2 * DIM, 2 * DIM), lambda i: (0, 0)))
  in_specs.append(pl.BlockSpec((2 * DIM, 2), lambda i: (0, 0)))

  return pl.pallas_call(
      _tc_body,
      grid=(bc // BB,),
      in_specs=in_specs,
      out_specs=pl.BlockSpec((BB // 2, 2), lambda i: (i, 0)),
      out_shape=jax.ShapeDtypeStruct((bc // 2, 2), jnp.float32),
  )(*([e_rows] * 16), *([r_rows] * 8), items_p, w1a2, w1b2, w2b)


def kernel(items, user_init_triple_set, item_potential_triple_set,
           user_potential_triple_set, item_origin_triple_set,
           entity_emb, relation_emb, W1, W2):
  sets = (user_init_triple_set, item_potential_triple_set,
          user_potential_triple_set, item_origin_triple_set)

  z = jnp.zeros((DIM, DIM), jnp.float32)
  w1a, w1b = W1[:DIM], W1[DIM:]
  w1a2 = jnp.concatenate(
      [jnp.concatenate([w1a, z], 1), jnp.concatenate([z, w1a], 1)], 0)
  w1b2 = jnp.concatenate(
      [jnp.concatenate([w1b, z], 1), jnp.concatenate([z, w1b], 1)], 0)
  zc = jnp.zeros((DIM, 1), jnp.float32)
  w2b = jnp.concatenate([jnp.concatenate([W2, zc], 0),
                         jnp.concatenate([zc, W2], 0)], 1)

  idt = sets[0].dtype
  items_rows = _sc_gather_items(entity_emb,
                                items.astype(idt).reshape(-1, 128))
  items_paired = items_rows.reshape(-1, 2 * DIM)

  offs = [0]
  for bc in CHUNKS:
    offs.append(offs[-1] + bc)

  e_chunk_idx = []
  r_chunk_idx = []
  for c, bc in enumerate(CHUNKS):
    sl = slice(offs[c], offs[c + 1])
    e_parts = []
    r_parts = []
    for ts in sets:
      e_parts += [ts[0, 0, sl].reshape(-1), ts[0, 1, sl].reshape(-1),
                  ts[2, 0, sl].reshape(-1), ts[2, 1, sl].reshape(-1)]
      r_parts += [ts[1, 0, sl].reshape(-1), ts[1, 1, sl].reshape(-1)]
    e_chunk_idx.append(jnp.concatenate(e_parts).reshape(-1, 128))
    r_chunk_idx.append(jnp.concatenate(r_parts).reshape(-1, 128))

  def gather_padded(table, idx2d, chunk):
    n = idx2d.shape[0] * 128
    gran = NW * 2 * chunk
    sup = -(-n // gran)
    npad = sup * gran - n
    if npad:
      idx2d = jnp.concatenate(
          [idx2d, jnp.zeros((npad // 128, 128), idx2d.dtype)])
    return _sc_gather(table, idx2d, chunk, chunk // 128, sup)

  e_rows = []
  r_rows = []
  for c, bc in enumerate(CHUNKS):
    ech = 512 if bc >= 1024 else (256 if bc >= 512 else 128)
    rch = 256 if bc >= 1024 else 128
    e_rows.append(gather_padded(entity_emb, e_chunk_idx[c], ech))
    r_rows.append(gather_padded(relation_emb, r_chunk_idx[c], rch))

  outs = []
  for c, bc in enumerate(CHUNKS):
    items_c = lax.dynamic_slice_in_dim(items_paired, offs[c] // 2,
                                       bc // 2, 0)
    outs.append(_tc_attention(e_rows[c].reshape(-1, 2 * DIM),
                              r_rows[c].reshape(-1, 2 * DIM),
                              items_c, w1a2, w1b2, w2b, bc))
  return jnp.concatenate(outs).reshape(B)
